# baseline (speedup 1.0000x reference)
import functools

import jax
import jax.numpy as jnp
from jax import lax
from jax.experimental import pallas as pl
from jax.experimental.pallas import tpu as pltpu
from jax.experimental.pallas import tpu_sc as plsc

NC = 2
NS = 16
NW = NC * NS
LANES = 16


def _sc_scores(cen_r, pos_r, neg_r, in_table, out_table, *, B, K, D):
    b_per_w = B // NW
    n_ib = b_per_w // 128
    n_nb = b_per_w * K // 128
    CB = 128
    n_chunks = b_per_w // CB
    rows_per_chunk = CB * K
    ndma = rows_per_chunk // 128
    ngrp = CB // LANES

    mesh = plsc.VectorSubcoreMesh(
        core_axis_name="c", subcore_axis_name="s",
        num_cores=NC, num_subcores=NS,
    )

    @functools.partial(
        pl.kernel,
        out_type=[
            jax.ShapeDtypeStruct((NW, n_chunks, ngrp, LANES), jnp.float32),
            jax.ShapeDtypeStruct((NW, n_chunks, K, ngrp, LANES), jnp.float32),
        ],
        mesh=mesh,
        compiler_params=pltpu.CompilerParams(
            needs_layout_passes=False, use_tc_tiling_on_sc=False),
        scratch_types=[
            pltpu.VMEM((n_ib, 128), jnp.int32),
            pltpu.VMEM((n_ib, 128), jnp.int32),
            pltpu.VMEM((n_nb, 128), jnp.int32),
            pltpu.VMEM((b_per_w, D), jnp.float32),
            pltpu.VMEM((b_per_w, D), jnp.float32),
            pltpu.VMEM((rows_per_chunk, D), jnp.float32),
            pltpu.VMEM((ngrp, LANES), jnp.float32),
            pltpu.VMEM((K, ngrp, LANES), jnp.float32),
            pltpu.SemaphoreType.DMA,
        ],
    )
    def scores_kernel(cen_hbm, pos_hbm, neg_hbm, in_hbm, out_hbm,
                      ps_out, ns_out,
                      cidx, pidx, nidx, vc, vo, vn, pos_acc, neg_acc, sem):
        wid = lax.axis_index("s") * NC + lax.axis_index("c")
        iota = lax.broadcasted_iota(jnp.int32, (LANES,), 0)
        zeros = jnp.zeros((LANES,), jnp.float32)

        pltpu.sync_copy(cen_hbm.at[wid], cidx)
        pltpu.sync_copy(pos_hbm.at[wid], pidx)
        pltpu.sync_copy(neg_hbm.at[wid], nidx)

        descs = []
        for j in range(n_ib):
            descs.append(pltpu.async_copy(
                in_hbm.at[cidx.at[j]], vc.at[pl.ds(j * 128, 128)], sem))
            descs.append(pltpu.async_copy(
                out_hbm.at[pidx.at[j]], vo.at[pl.ds(j * 128, 128)], sem))
        for dsc in descs:
            dsc.wait()

        def chunk_body(c, carry):
            nds = []
            for j in range(ndma):
                nds.append(pltpu.async_copy(
                    out_hbm.at[nidx.at[c * ndma + j]],
                    vn.at[pl.ds(j * 128, 128)], sem))
            for dsc in nds:
                dsc.wait()

            for g in range(ngrp):
                pos_acc[g, :] = zeros
            for k in range(K):
                for g in range(ngrp):
                    neg_acc[k, g, :] = zeros

            for g in range(ngrp):
                rows_b = c * CB + g * LANES + iota
                rows_n0 = (g * LANES + iota) * K

                def d_body(d, acc, rows_b=rows_b, rows_n0=rows_n0, g=g):
                    col = jnp.full((LANES,), d, jnp.int32)
                    vcc = plsc.load_gather(vc, [rows_b, col])
                    voc = plsc.load_gather(vo, [rows_b, col])
                    plsc.addupdate(pos_acc.at[g], vcc * voc)
                    for k in range(K):
                        vnc = plsc.load_gather(vn, [rows_n0 + k, col])
                        plsc.addupdate(neg_acc.at[k, g], vcc * vnc)
                    return acc

                lax.fori_loop(0, D, d_body, 0)

            pltpu.sync_copy(pos_acc, ps_out.at[wid, c])
            pltpu.sync_copy(neg_acc, ns_out.at[wid, c])
            return carry

        lax.fori_loop(0, n_chunks, chunk_body, 0)

    return scores_kernel(cen_r, pos_r, neg_r, in_table, out_table)


def _tc_loss(ps2d, ns2d, *, B):
    inv_b = 1.0 / float(B)

    def body(ps_ref, ns_ref, o_ref):
        def log_sig(x):
            return jnp.minimum(x, 0.0) - jnp.log(1.0 + jnp.exp(-jnp.abs(x)))

        pos_l = jnp.sum(log_sig(ps_ref[...]))
        neg_l = jnp.sum(log_sig(-ns_ref[...]))
        o_ref[...] = jnp.reshape(-(pos_l + neg_l) * inv_b, (1, 1))

    out = pl.pallas_call(
        body,
        out_shape=jax.ShapeDtypeStruct((1, 1), jnp.float32),
    )(ps2d, ns2d)
    return out[0, 0]


def kernel(centers, pos_contexts, neg_contexts, in_table, out_table):
    B = centers.shape[0]
    K = neg_contexts.shape[1]
    D = in_table.shape[1]
    b_per_w = B // NW

    cen_r = centers.astype(jnp.int32).reshape(NW, b_per_w // 128, 128)
    pos_r = pos_contexts.astype(jnp.int32).reshape(NW, b_per_w // 128, 128)
    neg_r = neg_contexts.astype(jnp.int32).reshape(NW, b_per_w * K // 128, 128)

    ps, ns = _sc_scores(cen_r, pos_r, neg_r, in_table, out_table,
                        B=B, K=K, D=D)
    return _tc_loss(ps.reshape(-1, 128), ns.reshape(-1, 128), B=B)


# speedup vs baseline: 1.1331x; 1.1331x over previous
import functools

import jax
import jax.numpy as jnp
from jax import lax
from jax.experimental import pallas as pl
from jax.experimental.pallas import tpu as pltpu
from jax.experimental.pallas import tpu_sc as plsc

NC = 2
NS = 16
NW = NC * NS
LANES = 16
KQ = 4


def _sc_scores(cen2d, pos2d, neg2d, in_table, out_table, *, B, K, D):
    b_per_w = B // NW
    n_ib = b_per_w // 128
    n_nb = b_per_w * K // 128
    CB = 64
    n_chunks = b_per_w // CB
    rows_per_chunk = CB * K
    ndma = rows_per_chunk // 128
    ngrp = CB // LANES

    mesh = plsc.VectorSubcoreMesh(
        core_axis_name="c", subcore_axis_name="s",
        num_cores=NC, num_subcores=NS,
    )

    @functools.partial(
        pl.kernel,
        out_type=[
            jax.ShapeDtypeStruct((NW, n_chunks, ngrp, LANES), jnp.float32),
            jax.ShapeDtypeStruct((NW, n_chunks, K, ngrp, LANES), jnp.float32),
        ],
        mesh=mesh,
        compiler_params=pltpu.CompilerParams(
            needs_layout_passes=False, use_tc_tiling_on_sc=False),
        scratch_types=[
            pltpu.VMEM((n_ib, 128), jnp.int32),
            pltpu.VMEM((n_ib, 128), jnp.int32),
            pltpu.VMEM((n_nb, 128), jnp.int32),
            pltpu.VMEM((2 * CB, D), jnp.float32),
            pltpu.VMEM((2 * CB, D), jnp.float32),
            pltpu.VMEM((2 * rows_per_chunk, D), jnp.float32),
            pltpu.VMEM((D, ngrp, LANES), jnp.float32),
            pltpu.VMEM((ngrp, LANES), jnp.float32),
            pltpu.VMEM((K, ngrp, LANES), jnp.float32),
            pltpu.SemaphoreType.DMA,
            pltpu.SemaphoreType.DMA,
        ],
    )
    def scores_kernel(cen_hbm, pos_hbm, neg_hbm, in_hbm, out_hbm,
                      ps_out, ns_out,
                      cidx, pidx, nidx, vcb, vob, vn, vcT, pos_acc, neg_acc,
                      sem_a, sem_b):
        wid = lax.axis_index("s") * NC + lax.axis_index("c")
        iota = lax.broadcasted_iota(jnp.int32, (LANES,), 0)

        pltpu.sync_copy(cen_hbm.at[pl.ds(wid * n_ib, n_ib)], cidx)
        pltpu.sync_copy(pos_hbm.at[pl.ds(wid * n_ib, n_ib)], pidx)
        pltpu.sync_copy(neg_hbm.at[pl.ds(wid * n_nb, n_nb)], nidx)

        def fire_chunk(c):
            half = jnp.bitwise_and(c, 1)
            crow = lax.div(c, 2)
            coff = jnp.bitwise_and(c, 1) * CB
            pltpu.async_copy(
                in_hbm.at[cidx.at[crow, pl.ds(coff, CB)]],
                vcb.at[pl.ds(half * CB, CB)], sem_a)
            pltpu.async_copy(
                out_hbm.at[pidx.at[crow, pl.ds(coff, CB)]],
                vob.at[pl.ds(half * CB, CB)], sem_a)
            for j in range(ndma):
                pltpu.async_copy(
                    out_hbm.at[nidx.at[c * ndma + j]],
                    vn.at[pl.ds(half * rows_per_chunk + j * 128, 128)], sem_b)

        fire_chunk(jnp.int32(0))

        def chunk_body(c, carry):
            half = jnp.bitwise_and(c, 1)
            nxt = c + 1

            @pl.when(nxt < n_chunks)
            def _fire():
                fire_chunk(nxt)

            pltpu.make_async_copy(
                in_hbm.at[pl.ds(0, CB)], vcb.at[pl.ds(half * CB, CB)],
                sem_a).wait()
            pltpu.make_async_copy(
                in_hbm.at[pl.ds(0, CB)], vob.at[pl.ds(half * CB, CB)],
                sem_a).wait()
            pltpu.make_async_copy(
                out_hbm.at[pl.ds(0, rows_per_chunk)],
                vn.at[pl.ds(half * rows_per_chunk, rows_per_chunk)],
                sem_b).wait()

            vn_base = half * rows_per_chunk

            def g_body(g, carry2):
                rows16 = half * CB + g * LANES + iota
                acc_p = jnp.zeros((LANES,), jnp.float32)
                for d in range(D):
                    col = jnp.full((LANES,), d, jnp.int32)
                    vcc = plsc.load_gather(vcb, [rows16, col])
                    voc = plsc.load_gather(vob, [rows16, col])
                    vcT[d, g, :] = vcc
                    acc_p = acc_p + vcc * voc
                pos_acc[g, :] = acc_p

                rows_n0 = vn_base + (g * LANES + iota) * K
                for kq in range(K // KQ):
                    rows_k = [rows_n0 + (kq * KQ + t) for t in range(KQ)]
                    accs = [jnp.zeros((LANES,), jnp.float32)] * KQ
                    for d in range(D):
                        col = jnp.full((LANES,), d, jnp.int32)
                        cv = vcT[d, g, :]
                        for t in range(KQ):
                            vnc = plsc.load_gather(vn, [rows_k[t], col])
                            accs[t] = accs[t] + cv * vnc
                    for t in range(KQ):
                        neg_acc[kq * KQ + t, g, :] = accs[t]
                return carry2

            lax.fori_loop(0, ngrp, g_body, 0)
            pltpu.sync_copy(pos_acc, ps_out.at[wid, c])
            pltpu.sync_copy(neg_acc, ns_out.at[wid, c])
            return carry

        lax.fori_loop(0, n_chunks, chunk_body, 0)

    return scores_kernel(cen2d, pos2d, neg2d, in_table, out_table)


def _tc_loss(ps2d, ns2d, *, B):
    inv_b = 1.0 / float(B)

    def body(ps_ref, ns_ref, o_ref):
        def log_sig(x):
            return jnp.minimum(x, 0.0) - jnp.log(1.0 + jnp.exp(-jnp.abs(x)))

        pos_l = jnp.sum(log_sig(ps_ref[...]))
        neg_l = jnp.sum(log_sig(-ns_ref[...]))
        o_ref[...] = jnp.reshape(-(pos_l + neg_l) * inv_b, (1, 1))

    out = pl.pallas_call(
        body,
        out_shape=jax.ShapeDtypeStruct((1, 1), jnp.float32),
    )(ps2d, ns2d)
    return out[0, 0]


def kernel(centers, pos_contexts, neg_contexts, in_table, out_table):
    B = centers.shape[0]
    K = neg_contexts.shape[1]
    D = in_table.shape[1]

    cen2d = centers.astype(jnp.int32).reshape(-1, 128)
    pos2d = pos_contexts.astype(jnp.int32).reshape(-1, 128)
    neg2d = neg_contexts.astype(jnp.int32).reshape(-1, 128)

    ps, ns = _sc_scores(cen2d, pos2d, neg2d, in_table, out_table,
                        B=B, K=K, D=D)
    return _tc_loss(ps.reshape(-1, 128), ns.reshape(-1, 128), B=B)
